# Initial kernel scaffold; baseline (speedup 1.0000x reference)
import jax
import jax.numpy as jnp
from jax.experimental import pallas as pl


def kernel(x, edge_index, W1, b1, W2, b2):
    raise NotImplementedError("write your pallas kernel here")


# speedup vs baseline: 35.3535x; 35.3535x over previous
import functools

import jax
import jax.numpy as jnp
from jax import lax
from jax.experimental import pallas as pl
from jax.experimental.pallas import tpu as pltpu
from jax.experimental.pallas import tpu_sc as plsc

NC = 2
NS = 16
NW = NC * NS
CHUNK = 128

_MESH = plsc.VectorSubcoreMesh(
    core_axis_name="c", subcore_axis_name="s", num_cores=NC, num_subcores=NS)
_SC_PARAMS = pltpu.CompilerParams(use_tc_tiling_on_sc=False)


def _worker_id():
  return lax.axis_index("s") * NC + lax.axis_index("c")


def _zero_fill(ref, nrows, ncols):
  per_row = ncols // 16

  def body(i, _):
    ref[i // per_row, pl.ds((i % per_row) * 16, 16)] = jnp.zeros(
        (16,), jnp.float32)
    return 0

  lax.fori_loop(0, nrows * per_row, body, 0)


def _deg_kernel(npad, cpw):
  sl = npad // NS

  @functools.partial(
      pl.kernel,
      out_type=jax.ShapeDtypeStruct((NC, npad), jnp.float32),
      mesh=_MESH,
      compiler_params=_SC_PARAMS,
      scratch_types=[
          pltpu.VMEM((cpw, CHUNK), jnp.int32),
          pltpu.VMEM((CHUNK,), jnp.float32),
          pltpu.VMEM((CHUNK,), jnp.float32),
          pltpu.VMEM_SHARED((npad,), jnp.float32),
      ],
  )
  def k(col_hbm, out_hbm, cidx, ones, zeros, acc):
    cid = lax.axis_index("c")
    sid = lax.axis_index("s")
    wid = _worker_id()

    def fill(i, _):
      ones[pl.ds(i * 16, 16)] = jnp.ones((16,), jnp.float32)
      zeros[pl.ds(i * 16, 16)] = jnp.zeros((16,), jnp.float32)
      return 0

    lax.fori_loop(0, CHUNK // 16, fill, 0)

    def zcopy(i, _):
      pltpu.sync_copy(zeros, acc.at[pl.ds(sid * sl + i * CHUNK, CHUNK)])
      return 0

    lax.fori_loop(0, sl // CHUNK, zcopy, 0)
    pltpu.sync_copy(col_hbm.at[wid], cidx)
    plsc.subcore_barrier()

    def body(j, _):
      pltpu.sync_copy(ones, acc.at[cidx.at[j]], add=True)
      return 0

    lax.fori_loop(0, cpw, body, 0)
    plsc.subcore_barrier()
    pltpu.sync_copy(acc.at[pl.ds(sid * sl, sl)],
                    out_hbm.at[cid, pl.ds(sid * sl, sl)])

  return k


def _agg_kernel(npad, cpw, h):
  sl = npad // NS

  @functools.partial(
      pl.kernel,
      out_type=jax.ShapeDtypeStruct((NC, npad, h), jnp.float32),
      mesh=_MESH,
      compiler_params=_SC_PARAMS,
      scratch_types=[
          pltpu.VMEM((cpw, CHUNK), jnp.int32),
          pltpu.VMEM((cpw, CHUNK), jnp.int32),
          pltpu.VMEM((CHUNK, h), jnp.float32),
          pltpu.VMEM_SHARED((npad, h), jnp.float32),
      ],
  )
  def k(u_hbm, row_hbm, col_hbm, out_hbm, ridx, cidx, gbuf, acc):
    cid = lax.axis_index("c")
    sid = lax.axis_index("s")
    wid = _worker_id()

    _zero_fill(gbuf, CHUNK, h)

    def zcopy(i, _):
      pltpu.sync_copy(gbuf, acc.at[pl.ds(sid * sl + i * CHUNK, CHUNK)])
      return 0

    lax.fori_loop(0, sl // CHUNK, zcopy, 0)
    pltpu.sync_copy(row_hbm.at[wid], ridx)
    pltpu.sync_copy(col_hbm.at[wid], cidx)
    plsc.subcore_barrier()

    def body(j, _):
      pltpu.sync_copy(u_hbm.at[ridx.at[j]], gbuf)
      pltpu.sync_copy(gbuf, acc.at[cidx.at[j]], add=True)
      return 0

    lax.fori_loop(0, cpw, body, 0)
    plsc.subcore_barrier()
    pltpu.sync_copy(acc.at[pl.ds(sid * sl, sl)],
                    out_hbm.at[cid, pl.ds(sid * sl, sl)])

  return k


def _agg1d_kernel(npad, cpw):
  sl = npad // NS

  @functools.partial(
      pl.kernel,
      out_type=jax.ShapeDtypeStruct((NC, npad), jnp.float32),
      mesh=_MESH,
      compiler_params=_SC_PARAMS,
      scratch_types=[
          pltpu.VMEM((cpw, CHUNK), jnp.int32),
          pltpu.VMEM((cpw, CHUNK), jnp.int32),
          pltpu.VMEM((CHUNK,), jnp.float32),
          pltpu.VMEM_SHARED((npad,), jnp.float32),
      ],
  )
  def k(u_hbm, row_hbm, col_hbm, out_hbm, ridx, cidx, gbuf, acc):
    cid = lax.axis_index("c")
    sid = lax.axis_index("s")
    wid = _worker_id()

    def fill(i, _):
      gbuf[pl.ds(i * 16, 16)] = jnp.zeros((16,), jnp.float32)
      return 0

    lax.fori_loop(0, CHUNK // 16, fill, 0)

    def zcopy(i, _):
      pltpu.sync_copy(gbuf, acc.at[pl.ds(sid * sl + i * CHUNK, CHUNK)])
      return 0

    lax.fori_loop(0, sl // CHUNK, zcopy, 0)
    pltpu.sync_copy(row_hbm.at[wid], ridx)
    pltpu.sync_copy(col_hbm.at[wid], cidx)
    plsc.subcore_barrier()

    def body(j, _):
      pltpu.sync_copy(u_hbm.at[ridx.at[j]], gbuf)
      pltpu.sync_copy(gbuf, acc.at[cidx.at[j]], add=True)
      return 0

    lax.fori_loop(0, cpw, body, 0)
    plsc.subcore_barrier()
    pltpu.sync_copy(acc.at[pl.ds(sid * sl, sl)],
                    out_hbm.at[cid, pl.ds(sid * sl, sl)])

  return k


def _tc_a(x, w1, degp_t, n):
  def body(x_ref, w1_ref, degp_ref, u_ref, dinv_ref):
    xw = jnp.dot(x_ref[...], w1_ref[...],
                 preferred_element_type=jnp.float32)
    deg = degp_ref[:n, 0:1] + degp_ref[:n, 1:2]
    dinv = lax.rsqrt(deg)
    u_ref[...] = xw * dinv
    dinv_ref[...] = dinv

  h = w1.shape[1]
  return pl.pallas_call(
      body,
      out_shape=(jax.ShapeDtypeStruct((n, h), jnp.float32),
                 jax.ShapeDtypeStruct((n, 1), jnp.float32)),
  )(x, w1, degp_t)


def _tc_b(accp, dinv, b1, w2, n):
  def body(accp_ref, dinv_ref, b1_ref, w2_ref, u2_ref):
    acc = accp_ref[0, :n, :] + accp_ref[1, :n, :]
    out1 = acc * dinv_ref[...] + b1_ref[...][None, :]
    hid = jnp.maximum(out1, 0.0)
    hw2 = jnp.dot(hid, w2_ref[...], preferred_element_type=jnp.float32)
    u2_ref[...] = hw2 * dinv_ref[...]

  return pl.pallas_call(
      body,
      out_shape=jax.ShapeDtypeStruct((n, 1), jnp.float32),
  )(accp, dinv, b1, w2)


def _tc_c(acc2p_t, dinv, b2, n):
  def body(acc2p_ref, dinv_ref, b2_ref, out_ref):
    q = acc2p_ref[:n, 0:1] + acc2p_ref[:n, 1:2]
    out_ref[...] = jax.nn.sigmoid(q * dinv_ref[...] + b2_ref[...][None, :])

  return pl.pallas_call(
      body,
      out_shape=jax.ShapeDtypeStruct((n, 1), jnp.float32),
  )(acc2p_t, dinv, b2)


def kernel(x, edge_index, W1, b1, W2, b2):
  n = x.shape[0]
  e = edge_index.shape[1]
  h = W1.shape[1]

  npad = ((n + NS * CHUNK - 1) // (NS * CHUNK)) * (NS * CHUNK)
  et = e + n
  cpw = (et + NW * CHUNK - 1) // (NW * CHUNK)
  ep = NW * cpw * CHUNK

  loop = jnp.arange(n, dtype=edge_index.dtype)
  pad = jnp.arange(ep - et, dtype=edge_index.dtype)
  row = jnp.concatenate([edge_index[0], loop, pad % n])
  col = jnp.concatenate([edge_index[1], loop, n + pad % (npad - n)])
  rowslab = row.reshape(NW, cpw, CHUNK)
  colslab = col.reshape(NW, cpw, CHUNK)

  degp = _deg_kernel(npad, cpw)(colslab)
  u, dinv = _tc_a(x, W1, degp.T, n)
  accp = _agg_kernel(npad, cpw, h)(u, rowslab, colslab)
  u2 = _tc_b(accp, dinv, b1, W2, n)
  acc2p = _agg1d_kernel(npad, cpw)(u2[:, 0], rowslab, colslab)
  out = _tc_c(acc2p.T, dinv, b2, n)
  return out[:, 0]


# speedup vs baseline: 54.2387x; 1.5342x over previous
import functools

import jax
import jax.numpy as jnp
from jax import lax
from jax.experimental import pallas as pl
from jax.experimental.pallas import tpu as pltpu
from jax.experimental.pallas import tpu_sc as plsc

NC = 2
NS = 16
NW = NC * NS
CHUNK = 128
K = 4

_MESH = plsc.VectorSubcoreMesh(
    core_axis_name="c", subcore_axis_name="s", num_cores=NC, num_subcores=NS)
_SC_PARAMS = pltpu.CompilerParams(use_tc_tiling_on_sc=False)


def _prologue(sid, wid, sl, zsrc, acc, idx_pairs):
  for i in range(sl // CHUNK):
    pltpu.sync_copy(zsrc, acc.at[pl.ds(sid * sl + i * CHUNK, CHUNK)])
  for hbm, vmem in idx_pairs:
    pltpu.sync_copy(hbm.at[wid], vmem)
  plsc.subcore_barrier()


def _epilogue(cid, sid, sl, acc, out_hbm):
  plsc.subcore_barrier()
  pltpu.sync_copy(acc.at[pl.ds(sid * sl, sl)],
                  out_hbm.at[cid, pl.ds(sid * sl, sl)])


def _deg_kernel(npad, cpw):
  sl = npad // NS

  @functools.partial(
      pl.kernel,
      out_type=jax.ShapeDtypeStruct((NC, npad), jnp.float32),
      mesh=_MESH,
      compiler_params=_SC_PARAMS,
      scratch_types=[
          pltpu.VMEM((cpw, CHUNK), jnp.int32),
          pltpu.VMEM((CHUNK,), jnp.float32),
          pltpu.VMEM((CHUNK,), jnp.float32),
          pltpu.VMEM_SHARED((npad,), jnp.float32),
          pltpu.SemaphoreType.DMA,
      ],
  )
  def k(col_hbm, out_hbm, cidx, ones, zeros, acc, ssem):
    cid = lax.axis_index("c")
    sid = lax.axis_index("s")
    wid = sid * NC + cid

    def fill(i, _):
      ones[pl.ds(i * 16, 16)] = jnp.ones((16,), jnp.float32)
      zeros[pl.ds(i * 16, 16)] = jnp.zeros((16,), jnp.float32)
      return 0

    lax.fori_loop(0, CHUNK // 16, fill, 0)
    _prologue(sid, wid, sl, zeros, acc, [(col_hbm, cidx)])

    pend = []
    for g in range(0, cpw, K):
      if len(pend) > K:
        for _ in range(K):
          pend.pop(0).wait()
      for j in range(g, g + K):
        pend.append(
            pltpu.async_copy(ones, acc.at[cidx.at[j]], ssem, add=True))
    for d in pend:
      d.wait()

    _epilogue(cid, sid, sl, acc, out_hbm)

  return k


def _agg_kernel(npad, cpw, h):
  sl = npad // NS
  ng = cpw // K
  assert cpw % K == 0
  gshape = (2, K, CHUNK) if h is None else (2, K, CHUNK, h)
  ashape = (npad,) if h is None else (npad, h)
  oshape = (NC, npad) if h is None else (NC, npad, h)

  @functools.partial(
      pl.kernel,
      out_type=jax.ShapeDtypeStruct(oshape, jnp.float32),
      mesh=_MESH,
      compiler_params=_SC_PARAMS,
      scratch_types=[
          pltpu.VMEM((cpw, CHUNK), jnp.int32),
          pltpu.VMEM((cpw, CHUNK), jnp.int32),
          pltpu.VMEM(gshape, jnp.float32),
          pltpu.VMEM_SHARED(ashape, jnp.float32),
          pltpu.SemaphoreType.DMA,
          pltpu.SemaphoreType.DMA,
      ],
  )
  def k(u_hbm, row_hbm, col_hbm, out_hbm, ridx, cidx, gbuf, acc, gsem, ssem):
    cid = lax.axis_index("c")
    sid = lax.axis_index("s")
    wid = sid * NC + cid

    zrow = gbuf.at[0, 0]
    if h is None:
      def zfill(i, _):
        gbuf[0, 0, pl.ds(i * 16, 16)] = jnp.zeros((16,), jnp.float32)
        return 0
      lax.fori_loop(0, CHUNK // 16, zfill, 0)
    else:
      per_row = h // 16
      def zfill(i, _):
        gbuf[0, 0, i // per_row, pl.ds((i % per_row) * 16, 16)] = jnp.zeros(
            (16,), jnp.float32)
        return 0
      lax.fori_loop(0, CHUNK * per_row, zfill, 0)
    _prologue(sid, wid, sl, zrow, acc,
              [(row_hbm, ridx), (col_hbm, cidx)])

    def fire_gathers(g):
      b = g % 2
      return [
          pltpu.async_copy(u_hbm.at[ridx.at[g * K + i]], gbuf.at[b, i], gsem)
          for i in range(K)
      ]

    def fire_scatters(g):
      b = g % 2
      return [
          pltpu.async_copy(gbuf.at[b, i], acc.at[cidx.at[g * K + i]], ssem,
                           add=True)
          for i in range(K)
      ]

    gathers = fire_gathers(0)
    scatters = []
    for g in range(ng):
      for d in scatters:
        d.wait()
      ngath = fire_gathers(g + 1) if g + 1 < ng else []
      for d in gathers:
        d.wait()
      scatters = fire_scatters(g)
      gathers = ngath
    for d in scatters:
      d.wait()

    _epilogue(cid, sid, sl, acc, out_hbm)

  return k


def _tc_a(x, w1, degp_t, n):
  def body(x_ref, w1_ref, degp_ref, u_ref, dinv_ref):
    xw = jnp.dot(x_ref[...], w1_ref[...],
                 preferred_element_type=jnp.float32)
    deg = degp_ref[:n, 0:1] + degp_ref[:n, 1:2]
    dinv = lax.rsqrt(deg)
    u_ref[...] = xw * dinv
    dinv_ref[...] = dinv

  h = w1.shape[1]
  return pl.pallas_call(
      body,
      out_shape=(jax.ShapeDtypeStruct((n, h), jnp.float32),
                 jax.ShapeDtypeStruct((n, 1), jnp.float32)),
  )(x, w1, degp_t)


def _tc_b(accp, dinv, b1, w2, n):
  def body(accp_ref, dinv_ref, b1_ref, w2_ref, u2_ref):
    acc = accp_ref[0, :n, :] + accp_ref[1, :n, :]
    out1 = acc * dinv_ref[...] + b1_ref[...][None, :]
    hid = jnp.maximum(out1, 0.0)
    hw2 = jnp.dot(hid, w2_ref[...], preferred_element_type=jnp.float32)
    u2_ref[...] = hw2 * dinv_ref[...]

  return pl.pallas_call(
      body,
      out_shape=jax.ShapeDtypeStruct((n, 1), jnp.float32),
  )(accp, dinv, b1, w2)


def _tc_c(acc2p_t, dinv, b2, n):
  def body(acc2p_ref, dinv_ref, b2_ref, out_ref):
    q = acc2p_ref[:n, 0:1] + acc2p_ref[:n, 1:2]
    out_ref[...] = jax.nn.sigmoid(q * dinv_ref[...] + b2_ref[...][None, :])

  return pl.pallas_call(
      body,
      out_shape=jax.ShapeDtypeStruct((n, 1), jnp.float32),
  )(acc2p_t, dinv, b2)


def kernel(x, edge_index, W1, b1, W2, b2):
  n = x.shape[0]
  e = edge_index.shape[1]
  h = W1.shape[1]

  npad = ((n + NS * CHUNK - 1) // (NS * CHUNK)) * (NS * CHUNK)
  et = e + n
  gsz = NW * CHUNK * K
  cpw = K * ((et + gsz - 1) // gsz)
  ep = NW * cpw * CHUNK

  loop = jnp.arange(n, dtype=edge_index.dtype)
  pad = jnp.arange(ep - et, dtype=edge_index.dtype)
  row = jnp.concatenate([edge_index[0], loop, pad % n])
  col = jnp.concatenate([edge_index[1], loop, n + pad % (npad - n)])
  rowslab = row.reshape(NW, cpw, CHUNK)
  colslab = col.reshape(NW, cpw, CHUNK)

  degp = _deg_kernel(npad, cpw)(colslab)
  u, dinv = _tc_a(x, W1, degp.T, n)
  accp = _agg_kernel(npad, cpw, h)(u, rowslab, colslab)
  u2 = _tc_b(accp, dinv, b1, W2, n)
  acc2p = _agg_kernel(npad, cpw, None)(u2[:, 0], rowslab, colslab)
  out = _tc_c(acc2p.T, dinv, b2, n)
  return out[:, 0]


# speedup vs baseline: 67.4054x; 1.2428x over previous
import functools

import jax
import jax.numpy as jnp
from jax import lax
from jax.experimental import pallas as pl
from jax.experimental.pallas import tpu as pltpu
from jax.experimental.pallas import tpu_sc as plsc

NC = 2
NS = 16
NW = NC * NS
CHUNK = 128
K = 4

_MESH = plsc.VectorSubcoreMesh(
    core_axis_name="c", subcore_axis_name="s", num_cores=NC, num_subcores=NS)
_SC_PARAMS = pltpu.CompilerParams(use_tc_tiling_on_sc=False)
_SC_VPARAMS = pltpu.CompilerParams(
    use_tc_tiling_on_sc=False, needs_layout_passes=False)


def _prologue(sid, wid, sl, zsrc, acc, idx_pairs):
  for i in range(sl // CHUNK):
    pltpu.sync_copy(zsrc, acc.at[pl.ds(sid * sl + i * CHUNK, CHUNK)])
  for hbm, vmem in idx_pairs:
    pltpu.sync_copy(hbm.at[wid], vmem)
  plsc.subcore_barrier()


def _epilogue(cid, sid, sl, acc, out_hbm):
  plsc.subcore_barrier()
  pltpu.sync_copy(acc.at[pl.ds(sid * sl, sl)],
                  out_hbm.at[cid, pl.ds(sid * sl, sl)])


def _deg_kernel(npad, cpw):
  sl = npad // NS

  @functools.partial(
      pl.kernel,
      out_type=jax.ShapeDtypeStruct((NC, npad), jnp.float32),
      mesh=_MESH,
      compiler_params=_SC_PARAMS,
      scratch_types=[
          pltpu.VMEM((cpw, CHUNK), jnp.int32),
          pltpu.VMEM((CHUNK,), jnp.float32),
          pltpu.VMEM((CHUNK,), jnp.float32),
          pltpu.VMEM_SHARED((npad,), jnp.float32),
          pltpu.SemaphoreType.DMA,
      ],
  )
  def k(col_hbm, out_hbm, cidx, ones, zeros, acc, ssem):
    cid = lax.axis_index("c")
    sid = lax.axis_index("s")
    wid = sid * NC + cid

    def fill(i, _):
      ones[pl.ds(i * 16, 16)] = jnp.ones((16,), jnp.float32)
      zeros[pl.ds(i * 16, 16)] = jnp.zeros((16,), jnp.float32)
      return 0

    lax.fori_loop(0, CHUNK // 16, fill, 0)
    _prologue(sid, wid, sl, zeros, acc, [(col_hbm, cidx)])

    pend = []
    for g in range(0, cpw, K):
      if len(pend) > K:
        for _ in range(K):
          pend.pop(0).wait()
      for j in range(g, g + K):
        pend.append(
            pltpu.async_copy(ones, acc.at[cidx.at[j]], ssem, add=True))
    for d in pend:
      d.wait()

    _epilogue(cid, sid, sl, acc, out_hbm)

  return k


NSETS = 5
AHEAD = 3


def _agg2d_kernel(npad, cpw, h):
  sl = npad // NS
  ng = cpw // K
  assert cpw % K == 0
  per_row = h // 16

  @functools.partial(
      pl.kernel,
      out_type=jax.ShapeDtypeStruct((NC, npad, h), jnp.float32),
      mesh=_MESH,
      compiler_params=_SC_PARAMS,
      scratch_types=[
          pltpu.VMEM((cpw, CHUNK), jnp.int32),
          pltpu.VMEM((cpw, CHUNK), jnp.int32),
          pltpu.VMEM((NSETS, K, CHUNK, h), jnp.float32),
          pltpu.VMEM_SHARED((npad, h), jnp.float32),
          pltpu.SemaphoreType.DMA,
          pltpu.SemaphoreType.DMA,
      ],
  )
  def k(u_hbm, row_hbm, col_hbm, out_hbm, ridx, cidx, gbuf, acc, gsem, ssem):
    cid = lax.axis_index("c")
    sid = lax.axis_index("s")
    wid = sid * NC + cid

    def zfill(i, _):
      gbuf[0, 0, i // per_row, pl.ds((i % per_row) * 16, 16)] = jnp.zeros(
          (16,), jnp.float32)
      return 0

    lax.fori_loop(0, CHUNK * per_row, zfill, 0)
    _prologue(sid, wid, sl, gbuf.at[0, 0], acc,
              [(row_hbm, ridx), (col_hbm, cidx)])

    def fire_gathers(g):
      b = g % NSETS
      return [
          pltpu.async_copy(u_hbm.at[ridx.at[g * K + i]], gbuf.at[b, i], gsem)
          for i in range(K)
      ]

    def fire_scatters(g):
      b = g % NSETS
      return [
          pltpu.async_copy(gbuf.at[b, i], acc.at[cidx.at[g * K + i]], ssem,
                           add=True)
          for i in range(K)
      ]

    pend_g = {}
    pend_s = {}
    for f in range(min(AHEAD, ng)):
      pend_g[f] = fire_gathers(f)
    for g in range(ng):
      f = g + AHEAD
      if f < ng:
        for d in pend_s.pop(f - NSETS, []):
          d.wait()
        pend_g[f] = fire_gathers(f)
      for d in pend_g.pop(g):
        d.wait()
      pend_s[g] = fire_scatters(g)
    for ds in pend_s.values():
      for d in ds:
        d.wait()

    _epilogue(cid, sid, sl, acc, out_hbm)

  return k


def _agg1d_kernel(npad, cpw, n):
  sl = npad // NS
  nbuf = 4

  @functools.partial(
      pl.kernel,
      out_type=jax.ShapeDtypeStruct((NC, npad), jnp.float32),
      mesh=_MESH,
      compiler_params=_SC_VPARAMS,
      scratch_types=[
          pltpu.VMEM((cpw, CHUNK), jnp.int32),
          pltpu.VMEM((cpw, CHUNK), jnp.int32),
          pltpu.VMEM((n,), jnp.float32),
          pltpu.VMEM((nbuf, CHUNK), jnp.float32),
          pltpu.VMEM_SHARED((npad,), jnp.float32),
          pltpu.SemaphoreType.DMA,
      ],
  )
  def k(u_hbm, row_hbm, col_hbm, out_hbm, ridx, cidx, u2t, gbuf, acc, ssem):
    cid = lax.axis_index("c")
    sid = lax.axis_index("s")
    wid = sid * NC + cid

    def zfill(i, _):
      gbuf[0, pl.ds(i * 16, 16)] = jnp.zeros((16,), jnp.float32)
      return 0

    lax.fori_loop(0, CHUNK // 16, zfill, 0)
    pltpu.sync_copy(u_hbm, u2t)
    _prologue(sid, wid, sl, gbuf.at[0], acc,
              [(row_hbm, ridx), (col_hbm, cidx)])

    pend = {}
    for j in range(cpw):
      b = j % nbuf
      for d in pend.pop(j - nbuf, []):
        d.wait()
      for t in range(CHUNK // 16):
        idxv = ridx[j, pl.ds(t * 16, 16)]
        gbuf[b, pl.ds(t * 16, 16)] = plsc.load_gather(u2t, [idxv])
      pend[j] = [
          pltpu.async_copy(gbuf.at[b], acc.at[cidx.at[j]], ssem, add=True)
      ]
    for ds in pend.values():
      for d in ds:
        d.wait()

    _epilogue(cid, sid, sl, acc, out_hbm)

  return k


def _tc_a(x, w1, degp_t, n):
  def body(x_ref, w1_ref, degp_ref, u_ref, dinv_ref):
    xw = jnp.dot(x_ref[...], w1_ref[...],
                 preferred_element_type=jnp.float32)
    deg = degp_ref[:n, 0:1] + degp_ref[:n, 1:2]
    dinv = lax.rsqrt(deg)
    u_ref[...] = xw * dinv
    dinv_ref[...] = dinv

  h = w1.shape[1]
  return pl.pallas_call(
      body,
      out_shape=(jax.ShapeDtypeStruct((n, h), jnp.float32),
                 jax.ShapeDtypeStruct((n, 1), jnp.float32)),
  )(x, w1, degp_t)


def _tc_b(accp, dinv, b1, w2, n):
  def body(accp_ref, dinv_ref, b1_ref, w2_ref, u2_ref):
    acc = accp_ref[0, :n, :] + accp_ref[1, :n, :]
    out1 = acc * dinv_ref[...] + b1_ref[...][None, :]
    hid = jnp.maximum(out1, 0.0)
    hw2 = jnp.dot(hid, w2_ref[...], preferred_element_type=jnp.float32)
    u2_ref[...] = hw2 * dinv_ref[...]

  return pl.pallas_call(
      body,
      out_shape=jax.ShapeDtypeStruct((n, 1), jnp.float32),
  )(accp, dinv, b1, w2)


def _tc_c(acc2p_t, dinv, b2, n):
  def body(acc2p_ref, dinv_ref, b2_ref, out_ref):
    q = acc2p_ref[:n, 0:1] + acc2p_ref[:n, 1:2]
    out_ref[...] = jax.nn.sigmoid(q * dinv_ref[...] + b2_ref[...][None, :])

  return pl.pallas_call(
      body,
      out_shape=jax.ShapeDtypeStruct((n, 1), jnp.float32),
  )(acc2p_t, dinv, b2)


def kernel(x, edge_index, W1, b1, W2, b2):
  n = x.shape[0]
  e = edge_index.shape[1]
  h = W1.shape[1]

  npad = ((n + NS * CHUNK - 1) // (NS * CHUNK)) * (NS * CHUNK)
  et = e + n
  gsz = NW * CHUNK * K
  cpw = K * ((et + gsz - 1) // gsz)
  ep = NW * cpw * CHUNK

  loop = jnp.arange(n, dtype=edge_index.dtype)
  pad = jnp.arange(ep - et, dtype=edge_index.dtype)
  row = jnp.concatenate([edge_index[0], loop, pad % n])
  col = jnp.concatenate([edge_index[1], loop, n + pad % (npad - n)])
  rowslab = row.reshape(NW, cpw, CHUNK)
  colslab = col.reshape(NW, cpw, CHUNK)

  degp = _deg_kernel(npad, cpw)(colslab)
  u, dinv = _tc_a(x, W1, degp.T, n)
  accp = _agg2d_kernel(npad, cpw, h)(u, rowslab, colslab)
  u2 = _tc_b(accp, dinv, b1, W2, n)
  acc2p = _agg1d_kernel(npad, cpw, n)(u2[:, 0], rowslab, colslab)
  out = _tc_c(acc2p.T, dinv, b2, n)
  return out[:, 0]


# speedup vs baseline: 89.4701x; 1.3273x over previous
import functools

import jax
import jax.numpy as jnp
import numpy as np
from jax import lax
from jax.experimental import pallas as pl
from jax.experimental.pallas import tpu as pltpu
from jax.experimental.pallas import tpu_sc as plsc

NC = 2
NS = 16
NW = NC * NS
CHUNK = 128
K = 4
NSETS = 5
AHEAD = 3

_MESH = plsc.VectorSubcoreMesh(
    core_axis_name="c", subcore_axis_name="s", num_cores=NC, num_subcores=NS)
_SC_PARAMS = pltpu.CompilerParams(use_tc_tiling_on_sc=False)
_SC_VPARAMS = pltpu.CompilerParams(
    use_tc_tiling_on_sc=False, needs_layout_passes=False)


def _prologue(sid, wid, sl, zsrc, acc, idx_pairs):
  for i in range(sl // CHUNK):
    pltpu.sync_copy(zsrc, acc.at[pl.ds(sid * sl + i * CHUNK, CHUNK)])
  for hbm, vmem in idx_pairs:
    pltpu.sync_copy(hbm, vmem)
  plsc.subcore_barrier()


def _epilogue(cid, sid, sl, acc, out_hbm):
  plsc.subcore_barrier()
  pltpu.sync_copy(acc.at[pl.ds(sid * sl, sl)],
                  out_hbm.at[cid, pl.ds(sid * sl, sl)])


def _deg_kernel(npad, cpw):
  sl = npad // NS

  @functools.partial(
      pl.kernel,
      out_type=jax.ShapeDtypeStruct((NC, npad), jnp.float32),
      mesh=_MESH,
      compiler_params=_SC_PARAMS,
      scratch_types=[
          pltpu.VMEM((cpw, CHUNK), jnp.int32),
          pltpu.VMEM((CHUNK,), jnp.float32),
          pltpu.VMEM((CHUNK,), jnp.float32),
          pltpu.VMEM_SHARED((npad,), jnp.float32),
          pltpu.SemaphoreType.DMA,
      ],
  )
  def k(slab_hbm, out_hbm, cidx, ones, zeros, acc, ssem):
    cid = lax.axis_index("c")
    sid = lax.axis_index("s")
    wid = sid * NC + cid

    def fill(i, _):
      ones[pl.ds(i * 16, 16)] = jnp.ones((16,), jnp.float32)
      zeros[pl.ds(i * 16, 16)] = jnp.zeros((16,), jnp.float32)
      return 0

    lax.fori_loop(0, CHUNK // 16, fill, 0)
    _prologue(sid, wid, sl, zeros, acc, [(slab_hbm.at[1, wid], cidx)])

    pend = []
    for g in range(0, cpw, K):
      if len(pend) > K:
        for _ in range(K):
          pend.pop(0).wait()
      for j in range(g, g + K):
        pend.append(
            pltpu.async_copy(ones, acc.at[cidx.at[j]], ssem, add=True))
    for d in pend:
      d.wait()

    _epilogue(cid, sid, sl, acc, out_hbm)

  return k


def _agg2d_kernel(npad, cpw, h):
  sl = npad // NS
  ng = cpw // K
  assert cpw % K == 0
  per_row = h // 16

  @functools.partial(
      pl.kernel,
      out_type=jax.ShapeDtypeStruct((NC, npad, h), jnp.float32),
      mesh=_MESH,
      compiler_params=_SC_PARAMS,
      scratch_types=[
          pltpu.VMEM((cpw, CHUNK), jnp.int32),
          pltpu.VMEM((cpw, CHUNK), jnp.int32),
          pltpu.VMEM((NSETS, K, CHUNK, h), jnp.float32),
          pltpu.VMEM_SHARED((npad, h), jnp.float32),
          pltpu.SemaphoreType.DMA,
          pltpu.SemaphoreType.DMA,
      ],
  )
  def k(u_hbm, slab_hbm, out_hbm, ridx, cidx, gbuf, acc, gsem, ssem):
    cid = lax.axis_index("c")
    sid = lax.axis_index("s")
    wid = sid * NC + cid

    def zfill(i, _):
      gbuf[0, 0, i // per_row, pl.ds((i % per_row) * 16, 16)] = jnp.zeros(
          (16,), jnp.float32)
      return 0

    lax.fori_loop(0, CHUNK * per_row, zfill, 0)
    _prologue(sid, wid, sl, gbuf.at[0, 0], acc,
              [(slab_hbm.at[0, wid], ridx), (slab_hbm.at[1, wid], cidx)])

    def fire_gathers(g):
      b = g % NSETS
      return [
          pltpu.async_copy(u_hbm.at[ridx.at[g * K + i]], gbuf.at[b, i], gsem)
          for i in range(K)
      ]

    def fire_scatters(g):
      b = g % NSETS
      return [
          pltpu.async_copy(gbuf.at[b, i], acc.at[cidx.at[g * K + i]], ssem,
                           add=True)
          for i in range(K)
      ]

    pend_g = {}
    pend_s = {}
    for f in range(min(AHEAD, ng)):
      pend_g[f] = fire_gathers(f)
    for g in range(ng):
      f = g + AHEAD
      if f < ng:
        for d in pend_s.pop(f - NSETS, []):
          d.wait()
        pend_g[f] = fire_gathers(f)
      for d in pend_g.pop(g):
        d.wait()
      pend_s[g] = fire_scatters(g)
    for ds in pend_s.values():
      for d in ds:
        d.wait()

    _epilogue(cid, sid, sl, acc, out_hbm)

  return k


def _agg1d_kernel(npad, cpw, n):
  sl = npad // NS
  nbuf = 4

  @functools.partial(
      pl.kernel,
      out_type=jax.ShapeDtypeStruct((NC, npad), jnp.float32),
      mesh=_MESH,
      compiler_params=_SC_VPARAMS,
      scratch_types=[
          pltpu.VMEM((cpw, CHUNK), jnp.int32),
          pltpu.VMEM((cpw, CHUNK), jnp.int32),
          pltpu.VMEM((n,), jnp.float32),
          pltpu.VMEM((nbuf, CHUNK), jnp.float32),
          pltpu.VMEM_SHARED((npad,), jnp.float32),
          pltpu.SemaphoreType.DMA,
      ],
  )
  def k(u_hbm, slab_hbm, out_hbm, ridx, cidx, u2t, gbuf, acc, ssem):
    cid = lax.axis_index("c")
    sid = lax.axis_index("s")
    wid = sid * NC + cid

    def zfill(i, _):
      gbuf[0, pl.ds(i * 16, 16)] = jnp.zeros((16,), jnp.float32)
      return 0

    lax.fori_loop(0, CHUNK // 16, zfill, 0)
    pltpu.sync_copy(u_hbm, u2t)
    _prologue(sid, wid, sl, gbuf.at[0], acc,
              [(slab_hbm.at[0, wid], ridx), (slab_hbm.at[1, wid], cidx)])

    pend = {}
    for j in range(cpw):
      b = j % nbuf
      for d in pend.pop(j - nbuf, []):
        d.wait()
      for t in range(CHUNK // 16):
        idxv = ridx[j, pl.ds(t * 16, 16)]
        gbuf[b, pl.ds(t * 16, 16)] = plsc.load_gather(u2t, [idxv])
      pend[j] = [
          pltpu.async_copy(gbuf.at[b], acc.at[cidx.at[j]], ssem, add=True)
      ]
    for ds in pend.values():
      for d in ds:
        d.wait()

    _epilogue(cid, sid, sl, acc, out_hbm)

  return k


def _tc_a(x, w1, degp, n):
  def body(x_ref, w1_ref, degp_ref, u_ref, dinv_ref):
    xw = jnp.dot(x_ref[...], w1_ref[...],
                 preferred_element_type=jnp.float32)
    deg = degp_ref[0:1, :n] + degp_ref[1:2, :n] + 1.0
    dinv_row = lax.rsqrt(deg)
    dinv_col = dinv_row.reshape(n, 1)
    u_ref[...] = xw * dinv_col
    dinv_ref[...] = dinv_row

  h = w1.shape[1]
  return pl.pallas_call(
      body,
      out_shape=(jax.ShapeDtypeStruct((n, h), jnp.float32),
                 jax.ShapeDtypeStruct((1, n), jnp.float32)),
  )(x, w1, degp)


def _tc_b(accp, u, dinv, b1, w2, n):
  def body(accp_ref, u_ref, dinv_ref, b1_ref, w2_ref, u2_ref):
    dinv_col = dinv_ref[...].reshape(n, 1)
    acc = accp_ref[0, :n, :] + accp_ref[1, :n, :] + u_ref[...]
    out1 = acc * dinv_col + b1_ref[...][None, :]
    hid = jnp.maximum(out1, 0.0)
    hw2 = jnp.dot(hid, w2_ref[...], preferred_element_type=jnp.float32)
    u2_ref[...] = (hw2 * dinv_col).reshape(1, n)

  return pl.pallas_call(
      body,
      out_shape=jax.ShapeDtypeStruct((1, n), jnp.float32),
  )(accp, u, dinv, b1, w2)


def _tc_c(acc2p, u2, dinv, b2, n):
  def body(acc2p_ref, u2_ref, dinv_ref, b2_ref, out_ref):
    q = acc2p_ref[0:1, :n] + acc2p_ref[1:2, :n] + u2_ref[...]
    out_ref[...] = jax.nn.sigmoid(q * dinv_ref[...] + b2_ref[0])

  return pl.pallas_call(
      body,
      out_shape=jax.ShapeDtypeStruct((1, n), jnp.float32),
  )(acc2p, u2, dinv, b2)


def kernel(x, edge_index, W1, b1, W2, b2):
  n = x.shape[0]
  e = edge_index.shape[1]
  h = W1.shape[1]

  npad = ((n + NS * CHUNK - 1) // (NS * CHUNK)) * (NS * CHUNK)
  gsz = NW * CHUNK * K
  cpw = K * ((e + gsz - 1) // gsz)
  ep = NW * cpw * CHUNK

  npr = np.arange(ep - e, dtype=np.int32)
  pad2 = jnp.asarray(np.stack([npr % n, n + npr % (npad - n)]))
  slab = jnp.concatenate([edge_index, pad2], axis=1).reshape(
      2, NW, cpw, CHUNK)

  degp = _deg_kernel(npad, cpw)(slab)
  u, dinv = _tc_a(x, W1, degp, n)
  accp = _agg2d_kernel(npad, cpw, h)(u, slab)
  u2 = _tc_b(accp, u, dinv, b1, W2, n)
  acc2p = _agg1d_kernel(npad, cpw, n)(u2[0], slab)
  out = _tc_c(acc2p, u2, dinv, b2, n)
  return out[0]


# speedup vs baseline: 98.3330x; 1.0991x over previous
import functools

import jax
import jax.numpy as jnp
import numpy as np
from jax import lax
from jax.experimental import pallas as pl
from jax.experimental.pallas import tpu as pltpu
from jax.experimental.pallas import tpu_sc as plsc

NC = 2
NS = 16
NW = NC * NS
CHUNK = 128
K = 4
NSETS = 5
AHEAD = 3

_MESH = plsc.VectorSubcoreMesh(
    core_axis_name="c", subcore_axis_name="s", num_cores=NC, num_subcores=NS)
_SC_PARAMS = pltpu.CompilerParams(use_tc_tiling_on_sc=False)
_SC_VPARAMS = pltpu.CompilerParams(
    use_tc_tiling_on_sc=False, needs_layout_passes=False)


def _prologue(sid, wid, sl, zsrc, acc, idx_pairs):
  for i in range(sl // CHUNK):
    pltpu.sync_copy(zsrc, acc.at[pl.ds(sid * sl + i * CHUNK, CHUNK)])
  for hbm, vmem in idx_pairs:
    pltpu.sync_copy(hbm, vmem)
  plsc.subcore_barrier()


def _epilogue(cid, sid, sl, acc, out_hbm):
  plsc.subcore_barrier()
  pltpu.sync_copy(acc.at[pl.ds(sid * sl, sl)],
                  out_hbm.at[cid, pl.ds(sid * sl, sl)])


def _deg_kernel(npad, cpw):
  sl = npad // NS

  @functools.partial(
      pl.kernel,
      out_type=jax.ShapeDtypeStruct((NC, npad), jnp.float32),
      mesh=_MESH,
      compiler_params=_SC_PARAMS,
      scratch_types=[
          pltpu.VMEM((cpw, CHUNK), jnp.int32),
          pltpu.VMEM((CHUNK,), jnp.float32),
          pltpu.VMEM((CHUNK,), jnp.float32),
          pltpu.VMEM_SHARED((npad,), jnp.float32),
          pltpu.SemaphoreType.DMA,
      ],
  )
  def k(slab_hbm, out_hbm, cidx, ones, zeros, acc, ssem):
    cid = lax.axis_index("c")
    sid = lax.axis_index("s")
    wid = sid * NC + cid

    def fill(i, _):
      ones[pl.ds(i * 16, 16)] = jnp.ones((16,), jnp.float32)
      zeros[pl.ds(i * 16, 16)] = jnp.zeros((16,), jnp.float32)
      return 0

    lax.fori_loop(0, CHUNK // 16, fill, 0)
    _prologue(sid, wid, sl, zeros, acc, [(slab_hbm.at[1, wid], cidx)])

    pend = []
    for g in range(0, cpw, K):
      if len(pend) > K:
        for _ in range(K):
          pend.pop(0).wait()
      for j in range(g, g + K):
        pend.append(
            pltpu.async_copy(ones, acc.at[cidx.at[j]], ssem, add=True))
    for d in pend:
      d.wait()

    _epilogue(cid, sid, sl, acc, out_hbm)

  return k


def _agg2d_kernel(npad, cpw, h):
  sl = npad // NS
  ng = cpw // K
  assert cpw % K == 0
  per_row = h // 16

  @functools.partial(
      pl.kernel,
      out_type=jax.ShapeDtypeStruct((NC, npad, h), jnp.float32),
      mesh=_MESH,
      compiler_params=_SC_PARAMS,
      scratch_types=[
          pltpu.VMEM((cpw, CHUNK), jnp.int32),
          pltpu.VMEM((cpw, CHUNK), jnp.int32),
          pltpu.VMEM((NSETS, K, CHUNK, h), jnp.float32),
          pltpu.VMEM_SHARED((npad, h), jnp.float32),
          pltpu.SemaphoreType.DMA,
          pltpu.SemaphoreType.DMA,
      ],
  )
  def k(u_hbm, slab_hbm, out_hbm, ridx, cidx, gbuf, acc, gsem, ssem):
    cid = lax.axis_index("c")
    sid = lax.axis_index("s")
    wid = sid * NC + cid

    def zfill(i, _):
      gbuf[0, 0, i // per_row, pl.ds((i % per_row) * 16, 16)] = jnp.zeros(
          (16,), jnp.float32)
      return 0

    lax.fori_loop(0, CHUNK * per_row, zfill, 0)
    _prologue(sid, wid, sl, gbuf.at[0, 0], acc,
              [(slab_hbm.at[0, wid], ridx), (slab_hbm.at[1, wid], cidx)])

    def fire_gathers(g):
      b = g % NSETS
      return [
          pltpu.async_copy(u_hbm.at[ridx.at[g * K + i]], gbuf.at[b, i], gsem)
          for i in range(K)
      ]

    def fire_scatters(g):
      b = g % NSETS
      return [
          pltpu.async_copy(gbuf.at[b, i], acc.at[cidx.at[g * K + i]], ssem,
                           add=True)
          for i in range(K)
      ]

    pend_g = {}
    pend_s = {}
    for f in range(min(AHEAD, ng)):
      pend_g[f] = fire_gathers(f)
    for g in range(ng):
      f = g + AHEAD
      if f < ng:
        for d in pend_s.pop(f - NSETS, []):
          d.wait()
        pend_g[f] = fire_gathers(f)
      for d in pend_g.pop(g):
        d.wait()
      pend_s[g] = fire_scatters(g)
    for ds in pend_s.values():
      for d in ds:
        d.wait()

    _epilogue(cid, sid, sl, acc, out_hbm)

  return k


def _agg1d_kernel(npad, cpw, n):
  sl = npad // NS
  nbuf = 4

  @functools.partial(
      pl.kernel,
      out_type=jax.ShapeDtypeStruct((NC, npad), jnp.float32),
      mesh=_MESH,
      compiler_params=_SC_VPARAMS,
      scratch_types=[
          pltpu.VMEM((cpw, CHUNK), jnp.int32),
          pltpu.VMEM((cpw, CHUNK), jnp.int32),
          pltpu.VMEM((n,), jnp.float32),
          pltpu.VMEM((nbuf, CHUNK), jnp.float32),
          pltpu.VMEM_SHARED((npad,), jnp.float32),
          pltpu.SemaphoreType.DMA,
      ],
  )
  def k(u_hbm, slab_hbm, out_hbm, ridx, cidx, u2t, gbuf, acc, ssem):
    cid = lax.axis_index("c")
    sid = lax.axis_index("s")
    wid = sid * NC + cid

    def zfill(i, _):
      gbuf[0, pl.ds(i * 16, 16)] = jnp.zeros((16,), jnp.float32)
      return 0

    lax.fori_loop(0, CHUNK // 16, zfill, 0)
    pltpu.sync_copy(u_hbm, u2t)
    _prologue(sid, wid, sl, gbuf.at[0], acc,
              [(slab_hbm.at[0, wid], ridx), (slab_hbm.at[1, wid], cidx)])

    pend = {}
    for j in range(cpw):
      b = j % nbuf
      for d in pend.pop(j - nbuf, []):
        d.wait()
      for t in range(CHUNK // 16):
        idxv = ridx[j, pl.ds(t * 16, 16)]
        gbuf[b, pl.ds(t * 16, 16)] = plsc.load_gather(u2t, [idxv])
      pend[j] = [
          pltpu.async_copy(gbuf.at[b], acc.at[cidx.at[j]], ssem, add=True)
      ]
    for ds in pend.values():
      for d in ds:
        d.wait()

    _epilogue(cid, sid, sl, acc, out_hbm)

  return k


def _expand_mat(p, h):
  r = lax.broadcasted_iota(jnp.int32, (p, p * h), 0)
  c = lax.broadcasted_iota(jnp.int32, (p, p * h), 1)
  return jnp.where(r == c // h, 1.0, 0.0).astype(jnp.float32)


def _blockdiag(w, p):
  d, h = w.shape
  t = jnp.tile(w, (p, p))
  r = lax.broadcasted_iota(jnp.int32, (p * d, p * h), 0)
  c = lax.broadcasted_iota(jnp.int32, (p * d, p * h), 1)
  return jnp.where(r // d == c // h, t, 0.0)


def _tc_a(x4, w1, degp4, n4, q4, p):
  def body(x_ref, w1_ref, degp_ref, u_ref, dinvq_ref):
    h = w1_ref.shape[1]
    w14 = _blockdiag(w1_ref[...], p)
    xw4 = jnp.dot(x_ref[...], w14,
                  preferred_element_type=jnp.float32)
    degq = degp_ref[0] + degp_ref[1] + 1.0
    dinvq = lax.rsqrt(degq)
    dinv4 = jnp.dot(dinvq[:n4, :], _expand_mat(p, h),
                    preferred_element_type=jnp.float32)
    u_ref[...] = xw4 * dinv4
    dinvq_ref[...] = dinvq

  return pl.pallas_call(
      body,
      out_shape=(jax.ShapeDtypeStruct((n4, 128), jnp.float32),
                 jax.ShapeDtypeStruct((q4, p), jnp.float32)),
  )(x4, w1, degp4)


def _tc_b(accp4, u4, dinvq, b1, w2, n4, p):
  def body(accp_ref, u_ref, dinvq_ref, b1_ref, w2_ref, u2_ref):
    h = b1_ref.shape[0]
    dq = dinvq_ref[:n4, :]
    dinv4 = jnp.dot(dq, _expand_mat(p, h),
                    preferred_element_type=jnp.float32)
    acc4 = accp_ref[0, :n4, :] + accp_ref[1, :n4, :] + u_ref[...]
    b1_4 = jnp.tile(b1_ref[...], p)
    out1 = acc4 * dinv4 + b1_4[None, :]
    hid = jnp.maximum(out1, 0.0)
    w24 = _blockdiag(w2_ref[...], p)
    hw2q = jnp.dot(hid, w24,
                   preferred_element_type=jnp.float32)
    u2_ref[...] = hw2q * dq

  return pl.pallas_call(
      body,
      out_shape=jax.ShapeDtypeStruct((n4, p), jnp.float32),
  )(accp4, u4, dinvq, b1, w2)


def _tc_c(acc2q, u2q, dinvq, b2, n4):
  def body(acc2_ref, u2_ref, dinvq_ref, b2_ref, out_ref):
    q = acc2_ref[0, :n4, :] + acc2_ref[1, :n4, :] + u2_ref[...]
    out_ref[...] = jax.nn.sigmoid(q * dinvq_ref[:n4, :] + b2_ref[0])

  return pl.pallas_call(
      body,
      out_shape=jax.ShapeDtypeStruct(u2q.shape, jnp.float32),
  )(acc2q, u2q, dinvq, b2)


def kernel(x, edge_index, W1, b1, W2, b2):
  n = x.shape[0]
  d = x.shape[1]
  e = edge_index.shape[1]
  h = W1.shape[1]

  npad = ((n + NS * CHUNK - 1) // (NS * CHUNK)) * (NS * CHUNK)
  gsz = NW * CHUNK * K
  cpw = K * ((e + gsz - 1) // gsz)
  ep = NW * cpw * CHUNK

  npr = np.arange(ep - e, dtype=np.int32)
  pad2 = jnp.asarray(np.stack([npr % n, n + npr % (npad - n)]))
  slab = jnp.concatenate([edge_index, pad2], axis=1).reshape(
      2, NW, cpw, CHUNK)

  p = 128 // h
  n4 = n // p
  q4 = npad // p

  degp = _deg_kernel(npad, cpw)(slab)
  u4, dinvq = _tc_a(x.reshape(n4, p * d), W1,
                    degp.reshape(NC, q4, p), n4, q4, p)
  accp = _agg2d_kernel(npad, cpw, h)(u4.reshape(n, h), slab)
  u2q = _tc_b(accp.reshape(NC, q4, 128), u4, dinvq, b1, W2, n4, p)
  acc2p = _agg1d_kernel(npad, cpw, n)(u2q.reshape(n), slab)
  outq = _tc_c(acc2p.reshape(NC, q4, p), u2q, dinvq, b2, n4)
  return outq.reshape(n)
